# baseline (speedup 1.0000x reference)
import functools

import jax
import jax.numpy as jnp
from jax import lax
from jax.experimental import pallas as pl
from jax.experimental.pallas import tpu as pltpu
from jax.experimental.pallas import tpu_sc as plsc

B, C_IN, H, W = 2, 128, 256, 256
C_OUT = 32
D, GH, GW = 64, 64, 64
NPIX = H * W
NVOX = D * GH * GW
NTOT = B * NVOX


_HB = 16


def _conv_body(x_ref, w_ref, b_ref, out_ref):
    x = x_ref[0]
    x2 = x.reshape(C_IN, _HB * W)
    acc = lax.dot_general(x2, w_ref[...],
                          (((0,), (1,)), ((), ())),
                          preferred_element_type=jnp.float32)
    out_ref[...] = acc + b_ref[...]


def _conv_call(grid2d, conv_w, conv_b):
    grid = (B, H // _HB)
    return pl.pallas_call(
        _conv_body,
        grid=grid,
        in_specs=[
            pl.BlockSpec((1, C_IN, _HB, W), lambda b, h: (b, 0, h, 0)),
            pl.BlockSpec((C_OUT, C_IN), lambda b, h: (0, 0)),
            pl.BlockSpec((1, C_OUT), lambda b, h: (0, 0)),
        ],
        out_specs=pl.BlockSpec((_HB * W, C_OUT),
                               lambda b, h: (b * (H // _HB) + h, 0)),
        out_shape=jax.ShapeDtypeStruct((B * NPIX, C_OUT), jnp.float32),
    )(grid2d, conv_w, conv_b.reshape(1, C_OUT))


_IR, _IC = NVOX // 128, 128


def _index_body(p_ref, l_ref, gidx_ref, mask_ref):
    b = pl.program_id(0)

    def rb(v):
        return v.astype(jnp.bfloat16).astype(jnp.float32)

    def project(n):
        zf = lax.shift_right_logical(n, 12).astype(jnp.float32)
        yf = jnp.bitwise_and(lax.shift_right_logical(n, 6), 63).astype(
            jnp.float32)
        xf = jnp.bitwise_and(n, 63).astype(jnp.float32)
        x = rb(xf + l_ref[b, 0])
        y = rb(yf + l_ref[b, 1])
        z = rb(zf + l_ref[b, 2])

        def proj(j):
            return (rb(p_ref[b, j, 0]) * x + rb(p_ref[b, j, 1]) * y
                    + rb(p_ref[b, j, 2]) * z + rb(p_ref[b, j, 3]))

        p0, p1, p2, p3 = proj(0), proj(1), proj(2), proj(3)
        fx = ((p0 / p3) / 2.0 + 0.5) * jnp.float32(W)
        fy = ((p1 / p3) / 2.0 + 0.5) * jnp.float32(H)
        px = fx.astype(jnp.int32)
        py = fy.astype(jnp.int32)
        valid = ((p2 >= 0.0) & (px >= 0) & (px < W) & (py >= 0) & (py < H))
        gidx = jnp.where(valid,
                         py * W + px,
                         jnp.bitwise_and(n, NPIX - 1)) + b * NPIX
        return gidx, valid

    r = lax.broadcasted_iota(jnp.int32, (_IR, _IC), 0)
    c = lax.broadcasted_iota(jnp.int32, (_IR, _IC), 1)
    p = r * _IC + c
    qv = _CHT // 4
    v = (jnp.bitwise_and(p, ~(_CHT - 1)) + jnp.bitwise_and(p, 3) * qv
         + jnp.bitwise_and(lax.shift_right_logical(p, 2), qv - 1))
    gidx_ref[0] = project(v)[0]
    mask_ref[0] = project(p)[1].astype(jnp.float32)


def _index_call(vpm, vsl):
    return pl.pallas_call(
        _index_body,
        grid=(B,),
        in_specs=[
            pl.BlockSpec((B, 4, 4), lambda b: (0, 0, 0),
                         memory_space=pltpu.SMEM),
            pl.BlockSpec((B, 3), lambda b: (0, 0),
                         memory_space=pltpu.SMEM),
        ],
        out_specs=[
            pl.BlockSpec((1, _IR, _IC), lambda b: (b, 0, 0)),
            pl.BlockSpec((1, _IR, _IC), lambda b: (b, 0, 0)),
        ],
        out_shape=[
            jax.ShapeDtypeStruct((B, _IR, _IC), jnp.int32),
            jax.ShapeDtypeStruct((B, _IR, _IC), jnp.float32),
        ],
    )(vpm, vsl)


_NC, _NS = 2, 16
_NW = _NC * _NS
_RPW = NTOT // _NW
_CH = 1024
_NCHUNK = _RPW // _CH


_NBUF = 3


def _gather_body(table_hbm, idx_hbm, out_hbm, idx_v,
                 rows0, rows1, rows2, gsem0, gsem1, gsem2,
                 osem0, osem1, osem2):
    wid = lax.axis_index("s") * _NC + lax.axis_index("c")
    base = wid * _RPW
    pltpu.sync_copy(idx_hbm.at[pl.ds(base, _RPW)], idx_v)
    rows = (rows0, rows1, rows2)
    gsem = (gsem0, gsem1, gsem2)
    osem = (osem0, osem1, osem2)

    def gather(i):
        return pltpu.async_copy(
            table_hbm.at[idx_v.at[pl.ds(i * _CH, _CH)]],
            rows[i % _NBUF], gsem[i % _NBUF])

    gops = [None] * _NCHUNK
    oops = [None] * _NCHUNK
    for k in range(_NBUF):
        gops[k] = gather(k)
    for i in range(_NCHUNK):
        gops[i].wait()
        oops[i] = pltpu.async_copy(
            rows[i % _NBUF], out_hbm.at[pl.ds(base + i * _CH, _CH)],
            osem[i % _NBUF])
        if i + _NBUF < _NCHUNK:
            oops[i].wait()
            gops[i + _NBUF] = gather(i + _NBUF)
    for i in range(_NCHUNK - _NBUF, _NCHUNK):
        oops[i].wait()


def _gather_call(table, gidx_flat):
    mesh = plsc.VectorSubcoreMesh(core_axis_name="c", subcore_axis_name="s")
    f = pl.kernel(
        _gather_body,
        out_type=jax.ShapeDtypeStruct((NTOT, C_OUT), jnp.float32),
        mesh=mesh,
        scratch_types=[
            pltpu.VMEM((_RPW,), jnp.int32),
            pltpu.VMEM((_CH, C_OUT), jnp.float32),
            pltpu.VMEM((_CH, C_OUT), jnp.float32),
            pltpu.VMEM((_CH, C_OUT), jnp.float32),
            pltpu.SemaphoreType.DMA,
            pltpu.SemaphoreType.DMA,
            pltpu.SemaphoreType.DMA,
            pltpu.SemaphoreType.DMA,
            pltpu.SemaphoreType.DMA,
            pltpu.SemaphoreType.DMA,
        ],
        compiler_params=pltpu.CompilerParams(use_tc_tiling_on_sc=False),
    )
    return f(table, gidx_flat)


_CHT = 4096
_NT = NTOT // _CHT


_ZB = _CHT // (GH * GW)


def _trans_body(g_ref, m_ref, out_ref):
    g = g_ref[...]
    cat = jnp.concatenate(
        [g[:, 32 * q:32 * (q + 1)] for q in range(4)], axis=0)
    t = cat.T * m_ref[0, 0][None, :]
    out_ref[...] = t.reshape(1, C_OUT, _ZB, GH, GW)


def _trans_call(gathered128, mask2d):
    cpb = NVOX // _CHT
    return pl.pallas_call(
        _trans_body,
        grid=(_NT,),
        in_specs=[
            pl.BlockSpec((_CHT // 4, 4 * C_OUT), lambda t: (t, 0)),
            pl.BlockSpec((1, 1, _CHT), lambda t: (t, 0, 0)),
        ],
        out_specs=pl.BlockSpec((1, C_OUT, _ZB, GH, GW),
                               lambda t: (t // cpb, 0, t % cpb, 0, 0)),
        out_shape=jax.ShapeDtypeStruct((B, C_OUT, D, GH, GW), jnp.float32),
    )(gathered128, mask2d)


def kernel(grid2d, voxel_projection_matrix, voxel_sample_location,
           conv_w, conv_b):
    table = _conv_call(grid2d, conv_w, conv_b)
    gidx, mask = _index_call(voxel_projection_matrix, voxel_sample_location)
    gathered = _gather_call(table, gidx.reshape(NTOT))
    return _trans_call(gathered.reshape(NTOT // 4, 4 * C_OUT),
                       mask.reshape(_NT, 1, _CHT))


# speedup vs baseline: 1.0704x; 1.0704x over previous
import functools

import jax
import jax.numpy as jnp
from jax import lax
from jax.experimental import pallas as pl
from jax.experimental.pallas import tpu as pltpu
from jax.experimental.pallas import tpu_sc as plsc

B, C_IN, H, W = 2, 128, 256, 256
C_OUT = 32
D, GH, GW = 64, 64, 64
NPIX = H * W
NVOX = D * GH * GW
NTOT = B * NVOX


_HB = 16


def _conv_body(x_ref, w_ref, b_ref, out_ref):
    x = x_ref[0]
    x2 = x.reshape(C_IN, _HB * W)
    acc = lax.dot_general(x2, w_ref[...],
                          (((0,), (1,)), ((), ())),
                          preferred_element_type=jnp.float32)
    out_ref[...] = acc + b_ref[...]


def _conv_call(grid2d, conv_w, conv_b):
    grid = (B, H // _HB)
    return pl.pallas_call(
        _conv_body,
        grid=grid,
        in_specs=[
            pl.BlockSpec((1, C_IN, _HB, W), lambda b, h: (b, 0, h, 0)),
            pl.BlockSpec((C_OUT, C_IN), lambda b, h: (0, 0)),
            pl.BlockSpec((1, C_OUT), lambda b, h: (0, 0)),
        ],
        out_specs=pl.BlockSpec((_HB * W, C_OUT),
                               lambda b, h: (b * (H // _HB) + h, 0)),
        out_shape=jax.ShapeDtypeStruct((B * NPIX, C_OUT), jnp.float32),
    )(grid2d, conv_w, conv_b.reshape(1, C_OUT))


_IR, _IC = NVOX // 128, 128


def _index_body(p_ref, l_ref, gidx_ref, mask_ref):
    b = pl.program_id(0)

    def rb(v):
        return v.astype(jnp.bfloat16).astype(jnp.float32)

    def project(n):
        zf = lax.shift_right_logical(n, 12).astype(jnp.float32)
        yf = jnp.bitwise_and(lax.shift_right_logical(n, 6), 63).astype(
            jnp.float32)
        xf = jnp.bitwise_and(n, 63).astype(jnp.float32)
        x = rb(xf + l_ref[b, 0])
        y = rb(yf + l_ref[b, 1])
        z = rb(zf + l_ref[b, 2])

        def proj(j):
            return (rb(p_ref[b, j, 0]) * x + rb(p_ref[b, j, 1]) * y
                    + rb(p_ref[b, j, 2]) * z + rb(p_ref[b, j, 3]))

        p0, p1, p2, p3 = proj(0), proj(1), proj(2), proj(3)
        fx = ((p0 / p3) / 2.0 + 0.5) * jnp.float32(W)
        fy = ((p1 / p3) / 2.0 + 0.5) * jnp.float32(H)
        px = fx.astype(jnp.int32)
        py = fy.astype(jnp.int32)
        valid = ((p2 >= 0.0) & (px >= 0) & (px < W) & (py >= 0) & (py < H))
        gidx = jnp.where(valid,
                         py * W + px,
                         jnp.bitwise_and(n, NPIX - 1)) + b * NPIX
        return gidx, valid

    r = lax.broadcasted_iota(jnp.int32, (_IR, _IC), 0)
    c = lax.broadcasted_iota(jnp.int32, (_IR, _IC), 1)
    p = r * _IC + c
    qv = _CHT // 4
    v = (jnp.bitwise_and(p, ~(_CHT - 1)) + jnp.bitwise_and(p, 3) * qv
         + jnp.bitwise_and(lax.shift_right_logical(p, 2), qv - 1))
    gidx_ref[0] = project(v)[0]
    mask_ref[0] = project(p)[1].astype(jnp.float32)


def _index_call(vpm, vsl):
    return pl.pallas_call(
        _index_body,
        grid=(B,),
        in_specs=[
            pl.BlockSpec((B, 4, 4), lambda b: (0, 0, 0),
                         memory_space=pltpu.SMEM),
            pl.BlockSpec((B, 3), lambda b: (0, 0),
                         memory_space=pltpu.SMEM),
        ],
        out_specs=[
            pl.BlockSpec((1, _IR, _IC), lambda b: (b, 0, 0)),
            pl.BlockSpec((1, _IR, _IC), lambda b: (b, 0, 0)),
        ],
        out_shape=[
            jax.ShapeDtypeStruct((B, _IR, _IC), jnp.int32),
            jax.ShapeDtypeStruct((B, _IR, _IC), jnp.float32),
        ],
    )(vpm, vsl)


_NC, _NS = 2, 16
_NW = _NC * _NS
_RPW = NTOT // _NW
_CH = 1024
_NCHUNK = _RPW // _CH


_NBUF = 3


def _gather_body(table_hbm, idx_hbm, out_hbm, idx_v,
                 rows0, rows1, rows2, gsem0, gsem1, gsem2,
                 osem0, osem1, osem2):
    wid = lax.axis_index("s") * _NC + lax.axis_index("c")
    base = wid * _RPW
    pltpu.sync_copy(idx_hbm.at[pl.ds(base, _RPW)], idx_v)
    rows = (rows0, rows1, rows2)
    gsem = (gsem0, gsem1, gsem2)
    osem = (osem0, osem1, osem2)

    def gather(i):
        return pltpu.async_copy(
            table_hbm.at[idx_v.at[pl.ds(i * _CH, _CH)]],
            rows[i % _NBUF], gsem[i % _NBUF])

    gops = [None] * _NCHUNK
    oops = [None] * _NCHUNK
    for k in range(_NBUF):
        gops[k] = gather(k)
    for i in range(_NCHUNK):
        gops[i].wait()
        oops[i] = pltpu.async_copy(
            rows[i % _NBUF], out_hbm.at[pl.ds(base + i * _CH, _CH)],
            osem[i % _NBUF])
        if i + _NBUF < _NCHUNK:
            oops[i].wait()
            gops[i + _NBUF] = gather(i + _NBUF)
    for i in range(_NCHUNK - _NBUF, _NCHUNK):
        oops[i].wait()


def _gather_call(table, gidx_flat):
    mesh = plsc.VectorSubcoreMesh(core_axis_name="c", subcore_axis_name="s")
    f = pl.kernel(
        _gather_body,
        out_type=jax.ShapeDtypeStruct((NTOT, C_OUT), jnp.float32),
        mesh=mesh,
        scratch_types=[
            pltpu.VMEM((_RPW,), jnp.int32),
            pltpu.VMEM((_CH, C_OUT), jnp.float32),
            pltpu.VMEM((_CH, C_OUT), jnp.float32),
            pltpu.VMEM((_CH, C_OUT), jnp.float32),
            pltpu.SemaphoreType.DMA,
            pltpu.SemaphoreType.DMA,
            pltpu.SemaphoreType.DMA,
            pltpu.SemaphoreType.DMA,
            pltpu.SemaphoreType.DMA,
            pltpu.SemaphoreType.DMA,
        ],
        compiler_params=pltpu.CompilerParams(use_tc_tiling_on_sc=False),
    )
    return f(table, gidx_flat)


_CHT = 16384
_NT = NTOT // _CHT


_ZB = _CHT // (GH * GW)


def _trans_body(g_ref, m_ref, out_ref):
    g = g_ref[...]
    cat = jnp.concatenate(
        [g[:, 32 * q:32 * (q + 1)] for q in range(4)], axis=0)
    t = cat.T * m_ref[0, 0][None, :]
    out_ref[...] = t.reshape(1, C_OUT, _ZB, GH, GW)


def _trans_call(gathered128, mask2d):
    cpb = NVOX // _CHT
    return pl.pallas_call(
        _trans_body,
        grid=(_NT,),
        in_specs=[
            pl.BlockSpec((_CHT // 4, 4 * C_OUT), lambda t: (t, 0)),
            pl.BlockSpec((1, 1, _CHT), lambda t: (t, 0, 0)),
        ],
        out_specs=pl.BlockSpec((1, C_OUT, _ZB, GH, GW),
                               lambda t: (t // cpb, 0, t % cpb, 0, 0)),
        out_shape=jax.ShapeDtypeStruct((B, C_OUT, D, GH, GW), jnp.float32),
    )(gathered128, mask2d)


def kernel(grid2d, voxel_projection_matrix, voxel_sample_location,
           conv_w, conv_b):
    table = _conv_call(grid2d, conv_w, conv_b)
    gidx, mask = _index_call(voxel_projection_matrix, voxel_sample_location)
    gathered = _gather_call(table, gidx.reshape(NTOT))
    return _trans_call(gathered.reshape(NTOT // 4, 4 * C_OUT),
                       mask.reshape(_NT, 1, _CHT))
